# baseline (speedup 1.0000x reference)
import functools

import jax
import jax.numpy as jnp
from jax import lax
from jax.experimental import pallas as pl
from jax.experimental.pallas import tpu as pltpu
from jax.experimental.pallas import tpu_sc as plsc

_PAD = 0


def _copy_body(pos_ref, out_ref):
    out_ref[0] = pos_ref[...]


def _make_sc_mask(N, S):
    info = plsc.get_sparse_core_info()
    NC, NS, L = info.num_cores, info.num_subcores, info.num_lanes
    NW = NC * NS
    cols_per_w = (N * S) // NW
    n_vec = cols_per_w // L
    mesh = plsc.VectorSubcoreMesh(core_axis_name="c", subcore_axis_name="s")

    @functools.partial(
        pl.kernel,
        mesh=mesh,
        out_type=jax.ShapeDtypeStruct((N, S), jnp.int32),
        scratch_types=[
            pltpu.VMEM((cols_per_w,), jnp.int32),
            pltpu.VMEM((cols_per_w,), jnp.int32),
        ],
    )
    def sc_mask(src_hbm, mask_out, sbuf, mbuf):
        wid = lax.axis_index("s") * NC + lax.axis_index("c")
        row = wid // (S // cols_per_w)
        col = (wid % (S // cols_per_w)) * cols_per_w
        pltpu.sync_copy(src_hbm.at[row, pl.ds(col, cols_per_w)], sbuf)

        def body(i, _):
            v = sbuf[pl.ds(i * L, L)]
            mbuf[pl.ds(i * L, L)] = jnp.where(v == _PAD, 1, 0).astype(jnp.int32)
            return 0

        lax.fori_loop(0, n_vec, body, 0)
        pltpu.sync_copy(mbuf, mask_out.at[row, pl.ds(col, cols_per_w)])

    return sc_mask


def kernel(src, pos_table):
    N, S = src.shape
    _, E = pos_table.shape
    BS = 2048

    pos_emb = pl.pallas_call(
        _copy_body,
        grid=(S // BS, N),
        in_specs=[pl.BlockSpec((BS, E), lambda j, i: (j, 0))],
        out_specs=pl.BlockSpec((1, BS, E), lambda j, i: (i, j, 0)),
        out_shape=jax.ShapeDtypeStruct((N, S, E), pos_table.dtype),
    )(pos_table)

    mask_i32 = _make_sc_mask(N, S)(src)
    return pos_emb, mask_i32.astype(jnp.bool_)


# speedup vs baseline: 1.4043x; 1.4043x over previous
import jax
import jax.numpy as jnp
from jax.experimental import pallas as pl
from jax.experimental.pallas import tpu as pltpu

_PAD = 0
_NCH = 8


def _body(src_ref, table_hbm, out_hbm, mask_ref, buf, rsem, wsem):
    N = out_hbm.shape[0]
    S = table_hbm.shape[0]
    ch = S // _NCH
    reads = [
        pltpu.make_async_copy(
            table_hbm.at[pl.ds(c * ch, ch)], buf.at[pl.ds(c * ch, ch)], rsem.at[c]
        )
        for c in range(_NCH)
    ]
    for r in reads:
        r.start()
    mask_ref[...] = src_ref[...] == _PAD
    writes = []
    for c in range(_NCH):
        reads[c].wait()
        for n in range(N):
            w = pltpu.make_async_copy(
                buf.at[pl.ds(c * ch, ch)],
                out_hbm.at[n, pl.ds(c * ch, ch)],
                wsem.at[c, n],
            )
            w.start()
            writes.append(w)
    for w in writes:
        w.wait()


def kernel(src, pos_table):
    N, S = src.shape
    _, E = pos_table.shape

    pos_emb, mask = pl.pallas_call(
        _body,
        in_specs=[
            pl.BlockSpec(memory_space=pltpu.VMEM),
            pl.BlockSpec(memory_space=pl.ANY),
        ],
        out_specs=(
            pl.BlockSpec(memory_space=pl.ANY),
            pl.BlockSpec(memory_space=pltpu.VMEM),
        ),
        out_shape=(
            jax.ShapeDtypeStruct((N, S, E), pos_table.dtype),
            jax.ShapeDtypeStruct((N, S), jnp.bool_),
        ),
        scratch_shapes=[
            pltpu.VMEM((S, E), pos_table.dtype),
            pltpu.SemaphoreType.DMA((_NCH,)),
            pltpu.SemaphoreType.DMA((_NCH, N)),
        ],
    )(src, pos_table)
    return pos_emb, mask
